# baseline (speedup 1.0000x reference)
import functools

import jax
import jax.numpy as jnp
from jax import lax
from jax.experimental import pallas as pl
from jax.experimental.pallas import tpu as pltpu
from jax.experimental.pallas import tpu_sc as plsc

NC = 2
NS = 16
NW = NC * NS
CHUNK = 80


def _sc_mesh():
    return plsc.VectorSubcoreMesh(
        core_axis_name="c", subcore_axis_name="s", num_cores=NC, num_subcores=NS
    )


def _make_degree_kernel(n_nodes, nchunk, d):
    @functools.partial(
        pl.kernel,
        out_type=jax.ShapeDtypeStruct((NC, n_nodes, d), jnp.float32),
        mesh=_sc_mesh(),
        scratch_types=[
            pltpu.VMEM((nchunk, CHUNK), jnp.int32),
            pltpu.VMEM((CHUNK, d), jnp.float32),
            pltpu.VMEM_SHARED((n_nodes, d), jnp.float32),
        ],
    )
    def deg_kernel(dstr_hbm, ones_hbm, zeros_hbm, out_hbm, dst_v, ones_v, acc_sh):
        c = lax.axis_index("c")
        s = lax.axis_index("s")
        wid = s * NC + c
        pltpu.sync_copy(dstr_hbm.at[wid], dst_v)
        pltpu.sync_copy(ones_hbm, ones_v)

        @pl.when(s == 0)
        def _():
            pltpu.sync_copy(zeros_hbm, acc_sh)

        plsc.subcore_barrier()

        def body(i, _):
            pltpu.sync_copy(ones_v, acc_sh.at[dst_v.at[i]], add=True)
            return 0

        lax.fori_loop(0, nchunk, body, 0)
        plsc.subcore_barrier()

        @pl.when(s == 0)
        def _():
            pltpu.sync_copy(acc_sh, out_hbm.at[c])

    return deg_kernel


NBUF = 3
SHIFT = 14
MASK = (1 << SHIFT) - 1


def _make_scatter_kernel(n_acc, d, nchunk):
    assert nchunk % NBUF == 0
    nblk = nchunk // NBUF

    @functools.partial(
        pl.kernel,
        out_type=jax.ShapeDtypeStruct((NC, n_acc, d), jnp.float32),
        mesh=_sc_mesh(),
        scratch_types=[
            pltpu.VMEM((nchunk, CHUNK), jnp.int32),
            [pltpu.VMEM((CHUNK, d), jnp.float32) for _ in range(NBUF)],
            [pltpu.VMEM((CHUNK,), jnp.int32) for _ in range(NBUF)],
            [pltpu.VMEM((CHUNK,), jnp.int32) for _ in range(NBUF)],
            pltpu.VMEM_SHARED((n_acc, d), jnp.float32),
            [pltpu.SemaphoreType.DMA for _ in range(NBUF)],
            [pltpu.SemaphoreType.DMA for _ in range(NBUF)],
        ],
    )
    def scat_kernel(
        g_hbm, pidx_hbm, zeros_hbm, out_hbm,
        pidx_v, rows, srcs, dsts, acc_sh, gsems, ssems,
    ):
        cax = lax.axis_index("c")
        sax = lax.axis_index("s")
        wid = sax * NC + cax
        pltpu.sync_copy(pidx_hbm.at[wid], pidx_v)

        @pl.when(sax == 0)
        def _():
            pltpu.sync_copy(zeros_hbm, acc_sh)

        plsc.subcore_barrier()

        def unpack(i, b):
            for k in range(CHUNK // 16):
                v = pidx_v[i, pl.ds(k * 16, 16)]
                srcs[b][pl.ds(k * 16, 16)] = lax.shift_right_logical(v, SHIFT)
                dsts[b][pl.ds(k * 16, 16)] = lax.bitwise_and(v, MASK)

        def wait_g(b):
            pltpu.make_async_copy(g_hbm.at[srcs[b]], rows[b], gsems[b]).wait()

        def wait_s(b):
            pltpu.make_async_copy(rows[b], acc_sh.at[dsts[b]], ssems[b]).wait()

        for b in range(NBUF - 1):
            unpack(b, b)
            pltpu.async_copy(g_hbm.at[srcs[b]], rows[b], gsems[b])
        for k in range(CHUNK // 16):
            dsts[NBUF - 1][pl.ds(k * 16, 16)] = jnp.full((16,), n_acc - 1, jnp.int32)
        pltpu.async_copy(rows[NBUF - 1], acc_sh.at[dsts[NBUF - 1]],
                         ssems[NBUF - 1], add=True)

        def slot(i, b):
            bm1 = (b + NBUF - 1) % NBUF
            wait_g(b)
            pltpu.async_copy(rows[b], acc_sh.at[dsts[b]], ssems[b], add=True)
            wait_s(bm1)
            unpack(i + NBUF - 1, bm1)
            pltpu.async_copy(g_hbm.at[srcs[bm1]], rows[bm1], gsems[bm1])

        def blk_body(blk, _):
            i0 = blk * NBUF
            for b in range(NBUF):
                slot(i0 + b, b)
            return 0

        lax.fori_loop(0, nblk - 1, blk_body, 0)
        i0 = (nblk - 1) * NBUF
        for b in range(NBUF):
            i = i0 + b
            bm1 = (b + NBUF - 1) % NBUF
            wait_g(b)
            pltpu.async_copy(rows[b], acc_sh.at[dsts[b]], ssems[b], add=True)
            wait_s(bm1)
            if i + NBUF - 1 < nchunk:
                unpack(i + NBUF - 1, bm1)
                pltpu.async_copy(g_hbm.at[srcs[bm1]], rows[bm1], gsems[bm1])
        wait_s(NBUF - 1)

        plsc.subcore_barrier()

        @pl.when(sax == 0)
        def _():
            pltpu.sync_copy(acc_sh, out_hbm.at[cax])

    return scat_kernel


def _dinv_from_cnt(cnt_blk):
    lanes = cnt_blk.shape[2]
    deg = 1.0 + (
        jnp.sum(cnt_blk[0], axis=1, keepdims=True)
        + jnp.sum(cnt_blk[1], axis=1, keepdims=True)
    ) * (1.0 / lanes)
    return lax.rsqrt(jnp.maximum(deg, 1.0))


def _prep_call(cnt, x, w1, blk):
    n, d = x.shape

    def body(cnt_ref, x_ref, w_ref, h_ref, g_ref):
        dinv = _dinv_from_cnt(cnt_ref[...])
        h = jnp.dot(x_ref[...], w_ref[...], preferred_element_type=jnp.float32)
        h_ref[...] = h
        g_ref[...] = h * dinv

    return pl.pallas_call(
        body,
        grid=(n // blk,),
        in_specs=[
            pl.BlockSpec((2, blk, d), lambda i: (0, i, 0)),
            pl.BlockSpec((blk, d), lambda i: (i, 0)),
            pl.BlockSpec((d, d), lambda i: (0, 0)),
        ],
        out_specs=[
            pl.BlockSpec((blk, d), lambda i: (i, 0)),
            pl.BlockSpec((blk, d), lambda i: (i, 0)),
        ],
        out_shape=[jax.ShapeDtypeStruct((n, d), jnp.float32)] * 2,
    )(cnt, x, w1)


def _mid_call(cnt, p, h, b, w_next, blk):
    n, d = h.shape

    def body(cnt_ref, p_ref, h_ref, b_ref, w_ref, h2_ref, g2_ref):
        dinv = _dinv_from_cnt(cnt_ref[...])
        agg = (p_ref[0] + p_ref[1]) * dinv + h_ref[...] * (dinv * dinv) + b_ref[...]
        y = jnp.maximum(agg, 0.0)
        h2 = jnp.dot(y, w_ref[...], preferred_element_type=jnp.float32)
        h2_ref[...] = h2
        g2_ref[...] = h2 * dinv

    return pl.pallas_call(
        body,
        grid=(n // blk,),
        in_specs=[
            pl.BlockSpec((2, blk, d), lambda i: (0, i, 0)),
            pl.BlockSpec((2, blk, d), lambda i: (0, i, 0)),
            pl.BlockSpec((blk, d), lambda i: (i, 0)),
            pl.BlockSpec((1, d), lambda i: (0, 0)),
            pl.BlockSpec((d, d), lambda i: (0, 0)),
        ],
        out_specs=[
            pl.BlockSpec((blk, d), lambda i: (i, 0)),
            pl.BlockSpec((blk, d), lambda i: (i, 0)),
        ],
        out_shape=[jax.ShapeDtypeStruct((n, d), jnp.float32)] * 2,
    )(cnt, p, h, b, w_next)


def _final_call(cnt, p, h, b3, wf1, bf1, wf2, bf2, wf3p, bf3p, n_classes, blk):
    n, d = h.shape

    def body(
        cnt_ref, p_ref, h_ref, b3_ref, wf1_ref, bf1_ref, wf2_ref, bf2_ref,
        wf3_ref, bf3_ref, out_ref,
    ):
        dinv = _dinv_from_cnt(cnt_ref[...])
        agg = (p_ref[0] + p_ref[1]) * dinv + h_ref[...] * (dinv * dinv) + b3_ref[...]
        y = jnp.maximum(agg, 0.0)
        z = jnp.maximum(
            jnp.dot(y, wf1_ref[...], preferred_element_type=jnp.float32)
            + bf1_ref[...], 0.0)
        z = jnp.maximum(
            jnp.dot(z, wf2_ref[...], preferred_element_type=jnp.float32)
            + bf2_ref[...], 0.0)
        z = (jnp.dot(z, wf3_ref[...], preferred_element_type=jnp.float32)
             + bf3_ref[...])
        col = lax.broadcasted_iota(jnp.int32, z.shape, 1)
        zm = jnp.where(col < n_classes, z, -1e30)
        m = jnp.max(zm, axis=1, keepdims=True)
        ssum = jnp.sum(jnp.exp(zm - m), axis=1, keepdims=True)
        out_ref[...] = z - m - jnp.log(ssum)

    wspec = pl.BlockSpec((d, d), lambda i: (0, 0))
    bspec = pl.BlockSpec((1, d), lambda i: (0, 0))
    return pl.pallas_call(
        body,
        grid=(n // blk,),
        in_specs=[
            pl.BlockSpec((2, blk, d), lambda i: (0, i, 0)),
            pl.BlockSpec((2, blk, d), lambda i: (0, i, 0)),
            pl.BlockSpec((blk, d), lambda i: (i, 0)),
            bspec, wspec, bspec, wspec, bspec, wspec, bspec,
        ],
        out_specs=pl.BlockSpec((blk, d), lambda i: (i, 0)),
        out_shape=jax.ShapeDtypeStruct((n, d), jnp.float32),
    )(cnt, p, h, b3, wf1, bf1, wf2, bf2, wf3p, bf3p)


def kernel(x, edge_index, W1, b1, W2, b2, W3, b3, Wf1, bf1, Wf2, bf2, Wf3, bf3):
    n, d = x.shape
    e = edge_index.shape[1]
    n_classes = Wf3.shape[1]
    assert e % NW == 0
    ept = e // NW
    block_edges = NBUF * CHUNK
    ept_pad = -(-ept // block_edges) * block_edges
    nchunk = ept_pad // CHUNK
    blk = 2048
    n_pad = -(-n // blk) * blk
    assert n_pad > n and n_pad <= (1 << SHIFT)

    padcol = jnp.full((NW, ept_pad - ept), n_pad - 1, jnp.int32)
    src_p = jnp.concatenate([edge_index[0].reshape(NW, ept), padcol], axis=1)
    dst_p = jnp.concatenate([edge_index[1].reshape(NW, ept), padcol], axis=1)
    packed = ((src_p << SHIFT) | dst_p).reshape(NW, nchunk, CHUNK)
    dstr = dst_p.reshape(NW, nchunk, CHUNK)
    zeros_acc = jnp.zeros((n_pad, d), jnp.float32)
    ones_c = jnp.ones((CHUNK, d), jnp.float32)
    xp = jnp.concatenate([x, jnp.zeros((n_pad - n, d), jnp.float32)])

    deg_k = _make_degree_kernel(n_pad, nchunk, d)
    cnt = deg_k(dstr, ones_c, zeros_acc)

    scat = _make_scatter_kernel(n_pad, d, nchunk)

    h1, g1 = _prep_call(cnt, xp, W1, blk)
    p1 = scat(g1, packed, zeros_acc)
    h2, g2 = _mid_call(cnt, p1, h1, b1.reshape(1, d), W2, blk)
    p2 = scat(g2, packed, zeros_acc)
    h3, g3 = _mid_call(cnt, p2, h2, b2.reshape(1, d), W3, blk)
    p3 = scat(g3, packed, zeros_acc)

    wf3p = jnp.pad(Wf3, ((0, 0), (0, d - n_classes)))
    bf3p = jnp.pad(bf3, (0, d - n_classes)).reshape(1, d)
    outp = _final_call(
        cnt, p3, h3, b3.reshape(1, d),
        Wf1, bf1.reshape(1, d), Wf2, bf2.reshape(1, d), wf3p, bf3p,
        n_classes, blk,
    )
    return outp[:n, :n_classes]


# speedup vs baseline: 1.0157x; 1.0157x over previous
import functools

import jax
import jax.numpy as jnp
from jax import lax
from jax.experimental import pallas as pl
from jax.experimental.pallas import tpu as pltpu
from jax.experimental.pallas import tpu_sc as plsc

NC = 2
NS = 16
NW = NC * NS
CHUNK = 80


def _sc_mesh():
    return plsc.VectorSubcoreMesh(
        core_axis_name="c", subcore_axis_name="s", num_cores=NC, num_subcores=NS
    )


def _make_degree_kernel(n_nodes, nchunk, d):
    @functools.partial(
        pl.kernel,
        out_type=jax.ShapeDtypeStruct((NC, n_nodes, d), jnp.float32),
        mesh=_sc_mesh(),
        scratch_types=[
            pltpu.VMEM((nchunk, CHUNK), jnp.int32),
            pltpu.VMEM((CHUNK, d), jnp.float32),
            pltpu.VMEM_SHARED((n_nodes, d), jnp.float32),
        ],
    )
    def deg_kernel(dstr_hbm, ones_hbm, zeros_hbm, out_hbm, dst_v, ones_v, acc_sh):
        c = lax.axis_index("c")
        s = lax.axis_index("s")
        wid = s * NC + c
        pltpu.sync_copy(dstr_hbm.at[wid], dst_v)
        pltpu.sync_copy(ones_hbm, ones_v)

        @pl.when(s == 0)
        def _():
            pltpu.sync_copy(zeros_hbm, acc_sh)

        plsc.subcore_barrier()

        def body(i, _):
            pltpu.sync_copy(ones_v, acc_sh.at[dst_v.at[i]], add=True)
            return 0

        lax.fori_loop(0, nchunk, body, 0)
        plsc.subcore_barrier()

        @pl.when(s == 0)
        def _():
            pltpu.sync_copy(acc_sh, out_hbm.at[c])

    return deg_kernel


NBUF = 3
SHIFT = 14
MASK = (1 << SHIFT) - 1


def _make_scatter_kernel(n_acc, d, nchunk):
    assert nchunk % NBUF == 0
    nblk = nchunk // NBUF

    @functools.partial(
        pl.kernel,
        out_type=jax.ShapeDtypeStruct((NC, n_acc, d), jnp.float32),
        mesh=_sc_mesh(),
        scratch_types=[
            pltpu.VMEM((nchunk, CHUNK), jnp.int32),
            [pltpu.VMEM((CHUNK, d), jnp.float32) for _ in range(NBUF)],
            [pltpu.VMEM((CHUNK,), jnp.int32) for _ in range(NBUF)],
            [pltpu.VMEM((CHUNK,), jnp.int32) for _ in range(NBUF)],
            pltpu.VMEM_SHARED((n_acc, d), jnp.float32),
            [pltpu.SemaphoreType.DMA for _ in range(NBUF)],
        ],
    )
    def scat_kernel(
        g_hbm, pidx_hbm, zeros_hbm, out_hbm,
        pidx_v, rows, srcs, dsts, acc_sh, gsems,
    ):
        cax = lax.axis_index("c")
        sax = lax.axis_index("s")
        wid = sax * NC + cax
        pltpu.sync_copy(pidx_hbm.at[wid], pidx_v)

        @pl.when(sax == 0)
        def _():
            pltpu.sync_copy(zeros_hbm, acc_sh)

        plsc.subcore_barrier()

        def unpack(i, b):
            for k in range(CHUNK // 16):
                v = pidx_v[i, pl.ds(k * 16, 16)]
                srcs[b][pl.ds(k * 16, 16)] = lax.shift_right_logical(v, SHIFT)
                dsts[b][pl.ds(k * 16, 16)] = lax.bitwise_and(v, MASK)

        for b in range(NBUF):
            unpack(b, b)
            pltpu.async_copy(g_hbm.at[srcs[b]], rows[b], gsems[b])

        def blk_body(blk, _):
            i0 = blk * NBUF
            for b in range(NBUF):
                pltpu.make_async_copy(g_hbm.at[srcs[b]], rows[b], gsems[b]).wait()
                pltpu.sync_copy(rows[b], acc_sh.at[dsts[b]], add=True)
                unpack(i0 + b + NBUF, b)
                pltpu.async_copy(g_hbm.at[srcs[b]], rows[b], gsems[b])
            return 0

        lax.fori_loop(0, nblk - 1, blk_body, 0)
        for b in range(NBUF):
            pltpu.make_async_copy(g_hbm.at[srcs[b]], rows[b], gsems[b]).wait()
            pltpu.sync_copy(rows[b], acc_sh.at[dsts[b]], add=True)

        plsc.subcore_barrier()

        @pl.when(sax == 0)
        def _():
            pltpu.sync_copy(acc_sh, out_hbm.at[cax])

    return scat_kernel


def _dinv_from_cnt(cnt_blk):
    lanes = cnt_blk.shape[2]
    deg = 1.0 + (
        jnp.sum(cnt_blk[0], axis=1, keepdims=True)
        + jnp.sum(cnt_blk[1], axis=1, keepdims=True)
    ) * (1.0 / lanes)
    return lax.rsqrt(jnp.maximum(deg, 1.0))


def _prep_call(cnt, x, w1, blk):
    n, d = x.shape

    def body(cnt_ref, x_ref, w_ref, h_ref, g_ref):
        dinv = _dinv_from_cnt(cnt_ref[...])
        h = jnp.dot(x_ref[...], w_ref[...], preferred_element_type=jnp.float32)
        h_ref[...] = h
        g_ref[...] = h * dinv

    return pl.pallas_call(
        body,
        grid=(n // blk,),
        in_specs=[
            pl.BlockSpec((2, blk, d), lambda i: (0, i, 0)),
            pl.BlockSpec((blk, d), lambda i: (i, 0)),
            pl.BlockSpec((d, d), lambda i: (0, 0)),
        ],
        out_specs=[
            pl.BlockSpec((blk, d), lambda i: (i, 0)),
            pl.BlockSpec((blk, d), lambda i: (i, 0)),
        ],
        out_shape=[jax.ShapeDtypeStruct((n, d), jnp.float32)] * 2,
    )(cnt, x, w1)


def _mid_call(cnt, p, h, b, w_next, blk):
    n, d = h.shape

    def body(cnt_ref, p_ref, h_ref, b_ref, w_ref, h2_ref, g2_ref):
        dinv = _dinv_from_cnt(cnt_ref[...])
        agg = (p_ref[0] + p_ref[1]) * dinv + h_ref[...] * (dinv * dinv) + b_ref[...]
        y = jnp.maximum(agg, 0.0)
        h2 = jnp.dot(y, w_ref[...], preferred_element_type=jnp.float32)
        h2_ref[...] = h2
        g2_ref[...] = h2 * dinv

    return pl.pallas_call(
        body,
        grid=(n // blk,),
        in_specs=[
            pl.BlockSpec((2, blk, d), lambda i: (0, i, 0)),
            pl.BlockSpec((2, blk, d), lambda i: (0, i, 0)),
            pl.BlockSpec((blk, d), lambda i: (i, 0)),
            pl.BlockSpec((1, d), lambda i: (0, 0)),
            pl.BlockSpec((d, d), lambda i: (0, 0)),
        ],
        out_specs=[
            pl.BlockSpec((blk, d), lambda i: (i, 0)),
            pl.BlockSpec((blk, d), lambda i: (i, 0)),
        ],
        out_shape=[jax.ShapeDtypeStruct((n, d), jnp.float32)] * 2,
    )(cnt, p, h, b, w_next)


def _final_call(cnt, p, h, b3, wf1, bf1, wf2, bf2, wf3p, bf3p, n_classes, blk):
    n, d = h.shape

    def body(
        cnt_ref, p_ref, h_ref, b3_ref, wf1_ref, bf1_ref, wf2_ref, bf2_ref,
        wf3_ref, bf3_ref, out_ref,
    ):
        dinv = _dinv_from_cnt(cnt_ref[...])
        agg = (p_ref[0] + p_ref[1]) * dinv + h_ref[...] * (dinv * dinv) + b3_ref[...]
        y = jnp.maximum(agg, 0.0)
        z = jnp.maximum(
            jnp.dot(y, wf1_ref[...], preferred_element_type=jnp.float32)
            + bf1_ref[...], 0.0)
        z = jnp.maximum(
            jnp.dot(z, wf2_ref[...], preferred_element_type=jnp.float32)
            + bf2_ref[...], 0.0)
        z = (jnp.dot(z, wf3_ref[...], preferred_element_type=jnp.float32)
             + bf3_ref[...])
        col = lax.broadcasted_iota(jnp.int32, z.shape, 1)
        zm = jnp.where(col < n_classes, z, -1e30)
        m = jnp.max(zm, axis=1, keepdims=True)
        ssum = jnp.sum(jnp.exp(zm - m), axis=1, keepdims=True)
        out_ref[...] = z - m - jnp.log(ssum)

    wspec = pl.BlockSpec((d, d), lambda i: (0, 0))
    bspec = pl.BlockSpec((1, d), lambda i: (0, 0))
    return pl.pallas_call(
        body,
        grid=(n // blk,),
        in_specs=[
            pl.BlockSpec((2, blk, d), lambda i: (0, i, 0)),
            pl.BlockSpec((2, blk, d), lambda i: (0, i, 0)),
            pl.BlockSpec((blk, d), lambda i: (i, 0)),
            bspec, wspec, bspec, wspec, bspec, wspec, bspec,
        ],
        out_specs=pl.BlockSpec((blk, d), lambda i: (i, 0)),
        out_shape=jax.ShapeDtypeStruct((n, d), jnp.float32),
    )(cnt, p, h, b3, wf1, bf1, wf2, bf2, wf3p, bf3p)


def kernel(x, edge_index, W1, b1, W2, b2, W3, b3, Wf1, bf1, Wf2, bf2, Wf3, bf3):
    n, d = x.shape
    e = edge_index.shape[1]
    n_classes = Wf3.shape[1]
    assert e % NW == 0
    ept = e // NW
    block_edges = NBUF * CHUNK
    ept_pad = -(-ept // block_edges) * block_edges
    nchunk = ept_pad // CHUNK
    blk = 2048
    n_pad = -(-n // blk) * blk
    assert n_pad > n and n_pad <= (1 << SHIFT)

    padcol = jnp.full((NW, ept_pad - ept), n_pad - 1, jnp.int32)
    src_p = jnp.concatenate([edge_index[0].reshape(NW, ept), padcol], axis=1)
    dst_p = jnp.concatenate([edge_index[1].reshape(NW, ept), padcol], axis=1)
    packed = ((src_p << SHIFT) | dst_p).reshape(NW, nchunk, CHUNK)
    dstr = dst_p.reshape(NW, nchunk, CHUNK)
    zeros_acc = jnp.zeros((n_pad, d), jnp.float32)
    ones_c = jnp.ones((CHUNK, d), jnp.float32)
    xp = jnp.concatenate([x, jnp.zeros((n_pad - n, d), jnp.float32)])

    deg_k = _make_degree_kernel(n_pad, nchunk, d)
    cnt = deg_k(dstr, ones_c, zeros_acc)

    scat = _make_scatter_kernel(n_pad, d, nchunk)

    h1, g1 = _prep_call(cnt, xp, W1, blk)
    p1 = scat(g1, packed, zeros_acc)
    h2, g2 = _mid_call(cnt, p1, h1, b1.reshape(1, d), W2, blk)
    p2 = scat(g2, packed, zeros_acc)
    h3, g3 = _mid_call(cnt, p2, h2, b2.reshape(1, d), W3, blk)
    p3 = scat(g3, packed, zeros_acc)

    wf3p = jnp.pad(Wf3, ((0, 0), (0, d - n_classes)))
    bf3p = jnp.pad(bf3, (0, d - n_classes)).reshape(1, d)
    outp = _final_call(
        cnt, p3, h3, b3.reshape(1, d),
        Wf1, bf1.reshape(1, d), Wf2, bf2.reshape(1, d), wf3p, bf3p,
        n_classes, blk,
    )
    return outp[:n, :n_classes]
